# baseline (speedup 1.0000x reference)
import functools

import jax
import jax.numpy as jnp
from jax import lax
from jax.experimental import pallas as pl
from jax.experimental.pallas import tpu as pltpu
from jax.experimental.pallas import tpu_sc as plsc

B, S, D, H = 1, 2048, 768, 12
DH = D // H
E, K, F, C = 64, 2, 512, 128
T = B * S
HALF = DH // 2

SB = 256
NC, NS = 2, 16
NW = NC * NS
TPW = T // NW
DUMMY = E * C

_f32 = jnp.float32
_i32 = jnp.int32
_bf16 = jnp.bfloat16


def _dot(a, b):
    return jnp.dot(a, b, preferred_element_type=_f32)


def _qkv_body(x_ref, anw_ref, wcat_ref, cos_ref, sin_ref, q_ref, k_ref, v_ref):
    xb = x_ref[...]
    hn = xb * lax.rsqrt(jnp.mean(xb * xb, axis=1, keepdims=True) + 1e-6)
    hn = hn * anw_ref[...]
    qkv = _dot(hn.astype(_bf16), wcat_ref[...])
    cosb = cos_ref[...]
    sinb = sin_ref[...]

    def rope(t):
        t1 = t[:, :HALF]
        t2 = t[:, HALF:]
        rot = jnp.concatenate([-t2, t1], axis=1)
        return t * cosb + rot * sinb

    for h in range(H):
        qh = qkv[:, h * DH:(h + 1) * DH]
        kh = qkv[:, D + h * DH:D + (h + 1) * DH]
        vh = qkv[:, 2 * D + h * DH:2 * D + (h + 1) * DH]
        q_ref[h] = rope(qh).astype(_bf16)
        k_ref[h] = rope(kh).astype(_bf16)
        v_ref[h] = vh.astype(_bf16)


def _qkv_call(x2, anw, wcat, cosf, sinf):
    hsd = jax.ShapeDtypeStruct((H, S, DH), _bf16)
    return pl.pallas_call(
        _qkv_body,
        grid=(S // SB,),
        in_specs=[
            pl.BlockSpec((SB, D), lambda s: (s, 0)),
            pl.BlockSpec((1, D), lambda s: (0, 0)),
            pl.BlockSpec((D, 3 * D), lambda s: (0, 0)),
            pl.BlockSpec((SB, DH), lambda s: (s, 0)),
            pl.BlockSpec((SB, DH), lambda s: (s, 0)),
        ],
        out_specs=[pl.BlockSpec((H, SB, DH), lambda s: (0, s, 0))] * 3,
        out_shape=[hsd, hsd, hsd],
    )(x2, anw, wcat, cosf, sinf)


KB = 512


def _attn_body(q_ref, k_ref, v_ref, o_ref):
    sq = pl.program_id(1)
    qb = q_ref[0]
    row = lax.broadcasted_iota(_i32, (SB, KB), 0) + sq * SB

    def body(kb, carry):
        m, l, acc = carry
        kblk = k_ref[0, pl.ds(kb * KB, KB), :]
        vblk = v_ref[0, pl.ds(kb * KB, KB), :]
        sb = lax.dot_general(qb, kblk, (((1,), (1,)), ((), ())),
                             preferred_element_type=_f32) * (1.0 / 8.0)
        col = lax.broadcasted_iota(_i32, (SB, KB), 1) + kb * KB
        sb = jnp.where(col <= row, sb, -1e30)
        mn = jnp.maximum(m, jnp.max(sb, axis=1, keepdims=True))
        p = jnp.exp(sb - mn)
        corr = jnp.exp(m - mn)
        l2 = l * corr + jnp.sum(p, axis=1, keepdims=True)
        acc2 = acc * corr + _dot(p.astype(_bf16), vblk)
        return mn, l2, acc2

    nkb = (sq * SB) // KB + 1
    m0 = jnp.full((SB, 1), -1e30, _f32)
    l0 = jnp.zeros((SB, 1), _f32)
    a0 = jnp.zeros((SB, DH), _f32)
    m, l, acc = lax.fori_loop(0, nkb, body, (m0, l0, a0))
    o_ref[0] = (acc / l).astype(_bf16)


def _attn_call(q, k, v):
    return pl.pallas_call(
        _attn_body,
        grid=(H, S // SB),
        in_specs=[
            pl.BlockSpec((1, SB, DH), lambda h, s: (h, s, 0)),
            pl.BlockSpec((1, S, DH), lambda h, s: (h, 0, 0)),
            pl.BlockSpec((1, S, DH), lambda h, s: (h, 0, 0)),
        ],
        out_specs=pl.BlockSpec((1, SB, DH), lambda h, s: (h, s, 0)),
        out_shape=jax.ShapeDtypeStruct((H, S, DH), _bf16),
    )(q, k, v)


def _projrouter_body(x_ref, o_ref, wo_ref, fnw_ref, rw_ref,
                     h_ref, zn_ref, w0_ref, w1_ref, s0_ref, s1_ref,
                     g0_ref, g1_ref, cnt_ref):
    @pl.when(pl.program_id(0) == 0)
    def _():
        cnt_ref[...] = jnp.zeros_like(cnt_ref)

    acc = x_ref[...]
    for h in range(H):
        acc = acc + _dot(o_ref[h], wo_ref[h])
    h_ref[...] = acc
    zn = acc * lax.rsqrt(jnp.mean(acc * acc, axis=1, keepdims=True) + 1e-6)
    zn = zn * fnw_ref[...]
    zn_ref[...] = zn
    lg = _dot(zn, rw_ref[...])
    io = lax.broadcasted_iota(_i32, (SB, E), 1)
    l1 = jnp.max(lg, axis=1, keepdims=True)
    e0 = jnp.min(jnp.where(lg == l1, io, E), axis=1, keepdims=True)
    lg2 = jnp.where(io == e0, -jnp.inf, lg)
    l2 = jnp.max(lg2, axis=1, keepdims=True)
    e1 = jnp.min(jnp.where(lg2 == l2, io, E), axis=1, keepdims=True)
    w0 = 1.0 / (1.0 + jnp.exp(l2 - l1))
    w1 = 1.0 - w0

    a0 = (io == e0).astype(_f32)
    a1 = (io == e1).astype(_f32)
    m2 = a0 + a1
    ri = lax.broadcasted_iota(_i32, (SB, SB), 0)
    ci = lax.broadcasted_iota(_i32, (SB, SB), 1)
    ltri = (ci <= ri).astype(_f32)
    incl = _dot(ltri, m2) + cnt_ref[0:1, :]
    cnt_ref[0:1, :] = incl[SB - 1:SB, :]

    pos0 = jnp.sum(incl * a0, axis=1, keepdims=True) - 1.0
    pos1 = jnp.sum(incl * a1, axis=1, keepdims=True) - 1.0
    keep0 = pos0 < C
    keep1 = pos1 < C
    p0i = pos0.astype(_i32)
    p1i = pos1.astype(_i32)
    pc0 = jnp.minimum(p0i, C - 1)
    pc1 = jnp.minimum(p1i, C - 1)

    def bcast(v):
        return jnp.broadcast_to(v, (SB, 128))

    w0_ref[...] = bcast(jnp.where(keep0, w0, 0.0))
    w1_ref[...] = bcast(jnp.where(keep1, w1, 0.0))
    s0_ref[...] = bcast(jnp.where(keep0, e0 * C + p0i, DUMMY))
    s1_ref[...] = bcast(jnp.where(keep1, e1 * C + p1i, DUMMY))
    g0_ref[...] = bcast(e0 * C + pc0)
    g1_ref[...] = bcast(e1 * C + pc1)


def _projrouter_call(x2, o, wo_r, fnw, rw):
    sd = jax.ShapeDtypeStruct((S, D), _f32)
    fs = jax.ShapeDtypeStruct((S, 128), _f32)
    is_ = jax.ShapeDtypeStruct((S, 128), _i32)
    return pl.pallas_call(
        _projrouter_body,
        grid=(S // SB,),
        in_specs=[
            pl.BlockSpec((SB, D), lambda s: (s, 0)),
            pl.BlockSpec((H, SB, DH), lambda s: (0, s, 0)),
            pl.BlockSpec((H, DH, D), lambda s: (0, 0, 0)),
            pl.BlockSpec((1, D), lambda s: (0, 0)),
            pl.BlockSpec((D, E), lambda s: (0, 0)),
        ],
        out_specs=([pl.BlockSpec((SB, D), lambda s: (s, 0))] * 2
                   + [pl.BlockSpec((SB, 128), lambda s: (s, 0))] * 6),
        out_shape=[sd, sd, fs, fs, is_, is_, is_, is_],
        scratch_shapes=[pltpu.VMEM((8, E), _f32)],
    )(x2, o, wo_r, fnw, rw)


@functools.cache
def _build_dispatch():
    mesh = plsc.VectorSubcoreMesh(core_axis_name="c", subcore_axis_name="s")

    @functools.partial(
        pl.kernel,
        mesh=mesh,
        out_type=jax.ShapeDtypeStruct((E * C + 128, D), _f32),
        scratch_types=[
            pltpu.VMEM((TPW,), _i32),
            pltpu.VMEM((TPW,), _i32),
            pltpu.VMEM((TPW, D), _f32),
            pltpu.SemaphoreType.DMA,
        ],
    )
    def disp(zn_hbm, s0_hbm, s1_hbm, out_hbm, i0_v, i1_v, rows_v, sem):
        wid = lax.axis_index("s") * NC + lax.axis_index("c")
        base = wid * TPW
        pltpu.sync_copy(s0_hbm.at[pl.ds(base, TPW)], i0_v)
        pltpu.sync_copy(s1_hbm.at[pl.ds(base, TPW)], i1_v)
        pltpu.sync_copy(zn_hbm.at[pl.ds(base, TPW)], rows_v)
        pltpu.async_copy(rows_v, out_hbm.at[i0_v], sem).wait()
        pltpu.async_copy(rows_v, out_hbm.at[i1_v], sem).wait()

    return disp


def _dispatch(zn, s0, s1):
    return _build_dispatch()(zn, s0, s1)


@functools.cache
def _build_combine():
    mesh = plsc.VectorSubcoreMesh(core_axis_name="c", subcore_axis_name="s")

    @functools.partial(
        pl.kernel,
        mesh=mesh,
        out_type=[jax.ShapeDtypeStruct((T, D), _f32),
                  jax.ShapeDtypeStruct((T, D), _f32)],
        scratch_types=[
            pltpu.VMEM((TPW,), _i32),
            pltpu.VMEM((TPW, D), _f32),
            pltpu.SemaphoreType.DMA,
        ],
    )
    def comb(oute_hbm, g0_hbm, g1_hbm, y0_hbm, y1_hbm, i_v, buf_v, sem):
        wid = lax.axis_index("s") * NC + lax.axis_index("c")
        base = wid * TPW
        pltpu.sync_copy(g0_hbm.at[pl.ds(base, TPW)], i_v)
        pltpu.async_copy(oute_hbm.at[i_v], buf_v, sem).wait()
        pltpu.sync_copy(buf_v, y0_hbm.at[pl.ds(base, TPW)])
        pltpu.sync_copy(g1_hbm.at[pl.ds(base, TPW)], i_v)
        pltpu.async_copy(oute_hbm.at[i_v], buf_v, sem).wait()
        pltpu.sync_copy(buf_v, y1_hbm.at[pl.ds(base, TPW)])

    return comb


def _combine(oute, g0, g1):
    return _build_combine()(oute, g0, g1)


def _ffn_body(in_ref, w1_ref, w3_ref, w2_ref, out_ref):
    xin = in_ref[...].astype(_bf16)
    g = _dot(xin, w1_ref[0].astype(_bf16))
    u = _dot(xin, w3_ref[0].astype(_bf16))
    a = g * (1.0 / (1.0 + jnp.exp(-g))) * u
    out_ref[...] = _dot(a.astype(_bf16), w2_ref[0].astype(_bf16))


def _ffn_call(ein, w1, w3, w2):
    return pl.pallas_call(
        _ffn_body,
        grid=(E,),
        in_specs=[
            pl.BlockSpec((C, D), lambda e: (e, 0)),
            pl.BlockSpec((1, D, F), lambda e: (e, 0, 0)),
            pl.BlockSpec((1, D, F), lambda e: (e, 0, 0)),
            pl.BlockSpec((1, F, D), lambda e: (e, 0, 0)),
        ],
        out_specs=pl.BlockSpec((C, D), lambda e: (e, 0)),
        out_shape=jax.ShapeDtypeStruct((E * C, D), _f32),
    )(ein, w1, w3, w2)


def _final_body(h_ref, y0_ref, y1_ref, w0_ref, w1_ref, o_ref):
    o_ref[...] = (h_ref[...]
                  + w0_ref[:, 0:1] * y0_ref[...]
                  + w1_ref[:, 0:1] * y1_ref[...])


def _final_call(h2, y0, y1, w0o, w1o):
    return pl.pallas_call(
        _final_body,
        grid=(S // SB,),
        in_specs=[
            pl.BlockSpec((SB, D), lambda s: (s, 0)),
            pl.BlockSpec((SB, D), lambda s: (s, 0)),
            pl.BlockSpec((SB, D), lambda s: (s, 0)),
            pl.BlockSpec((SB, 128), lambda s: (s, 0)),
            pl.BlockSpec((SB, 128), lambda s: (s, 0)),
        ],
        out_specs=pl.BlockSpec((SB, D), lambda s: (s, 0)),
        out_shape=jax.ShapeDtypeStruct((S, D), _f32),
    )(h2, y0, y1, w0o, w1o)


def kernel(x, attn_norm_w, Wq, Wk, Wv, Wo, ffn_norm_w, router_w, W1, W3, W2):
    x2 = x.reshape(S, D)
    wcat = jnp.concatenate([Wq, Wk, Wv], axis=1).astype(_bf16)
    wo_r = Wo.reshape(H, DH, D).astype(_bf16)
    anw = attn_norm_w.reshape(1, D)
    fnw = ffn_norm_w.reshape(1, D)

    inv_freq = 1.0 / (10000.0 ** (jnp.arange(0, HALF, dtype=_f32) / HALF))
    t = jnp.arange(S, dtype=_f32)
    freqs = jnp.outer(t, inv_freq)
    cosf = jnp.concatenate([jnp.cos(freqs)] * 2, axis=1)
    sinf = jnp.concatenate([jnp.sin(freqs)] * 2, axis=1)

    q, k, v = _qkv_call(x2, anw, wcat, cosf, sinf)
    o = _attn_call(q, k, v)
    (h2, zn, w0o, w1o, s0o, s1o, g0o, g1o) = _projrouter_call(
        x2, o, wo_r, fnw, router_w)

    s0 = s0o[:, 0]
    s1 = s1o[:, 0]
    g0 = g0o[:, 0]
    g1 = g1o[:, 0]

    ein = _dispatch(zn, s0, s1)
    oute = _ffn_call(ein, W1, W3, W2)
    y0, y1 = _combine(oute, g0, g1)

    out = _final_call(h2, y0, y1, w0o, w1o)
    return out.reshape(B, S, D)


# speedup vs baseline: 1.0967x; 1.0967x over previous
import functools

import jax
import jax.numpy as jnp
from jax import lax
from jax.experimental import pallas as pl
from jax.experimental.pallas import tpu as pltpu
from jax.experimental.pallas import tpu_sc as plsc

B, S, D, H = 1, 2048, 768, 12
DH = D // H
E, K, F, C = 64, 2, 512, 128
T = B * S
HALF = DH // 2

SB = 256
NC, NS = 2, 16
NW = NC * NS
TPW = T // NW
DUMMY = E * C

_f32 = jnp.float32
_i32 = jnp.int32
_bf16 = jnp.bfloat16


def _dot(a, b):
    return jnp.dot(a, b, preferred_element_type=_f32)


def _qkv_body(x_ref, anw_ref, wcat_ref, cos_ref, sin_ref, q_ref, k_ref, v_ref):
    xb = x_ref[...]
    hn = xb * lax.rsqrt(jnp.mean(xb * xb, axis=1, keepdims=True) + 1e-6)
    hn = hn * anw_ref[...]
    qkv = _dot(hn.astype(_bf16), wcat_ref[...])
    cosb = cos_ref[...]
    sinb = sin_ref[...]

    def rope(t):
        t1 = t[:, :HALF]
        t2 = t[:, HALF:]
        rot = jnp.concatenate([-t2, t1], axis=1)
        return t * cosb + rot * sinb

    for h in range(H):
        qh = qkv[:, h * DH:(h + 1) * DH]
        kh = qkv[:, D + h * DH:D + (h + 1) * DH]
        vh = qkv[:, 2 * D + h * DH:2 * D + (h + 1) * DH]
        q_ref[h] = rope(qh).astype(_bf16)
        k_ref[h] = rope(kh).astype(_bf16)
        v_ref[h] = vh.astype(_bf16)


def _qkv_call(x2, anw, wcat, cosf, sinf):
    hsd = jax.ShapeDtypeStruct((H, S, DH), _bf16)
    return pl.pallas_call(
        _qkv_body,
        grid=(S // SB,),
        in_specs=[
            pl.BlockSpec((SB, D), lambda s: (s, 0)),
            pl.BlockSpec((1, D), lambda s: (0, 0)),
            pl.BlockSpec((D, 3 * D), lambda s: (0, 0)),
            pl.BlockSpec((SB, DH), lambda s: (s, 0)),
            pl.BlockSpec((SB, DH), lambda s: (s, 0)),
        ],
        out_specs=[pl.BlockSpec((H, SB, DH), lambda s: (0, s, 0))] * 3,
        out_shape=[hsd, hsd, hsd],
    )(x2, anw, wcat, cosf, sinf)


def _attn_body(q_ref, k_ref, v_ref, o_ref, *, skd, sq_off):
    sq = pl.program_id(1) + sq_off
    qb = q_ref[0]
    kb = k_ref[0]
    sc = lax.dot_general(qb, kb, (((1,), (1,)), ((), ())),
                         preferred_element_type=_f32) * (1.0 / 8.0)
    row = lax.broadcasted_iota(_i32, (SB, skd), 0) + sq * SB
    col = lax.broadcasted_iota(_i32, (SB, skd), 1)
    sc = jnp.where(col <= row, sc, -1e30)
    m = jnp.max(sc, axis=1, keepdims=True)
    p = jnp.exp(sc - m)
    l = jnp.sum(p, axis=1, keepdims=True)
    o_ref[0] = (_dot(p.astype(_bf16), v_ref[0]) / l).astype(_bf16)


def _attn_half_call(q, k, v, *, skd, sq_off, nq):
    return pl.pallas_call(
        functools.partial(_attn_body, skd=skd, sq_off=sq_off),
        grid=(H, nq),
        in_specs=[
            pl.BlockSpec((1, SB, DH), lambda h, s: (h, s + sq_off, 0)),
            pl.BlockSpec((1, skd, DH), lambda h, s: (h, 0, 0)),
            pl.BlockSpec((1, skd, DH), lambda h, s: (h, 0, 0)),
        ],
        out_specs=pl.BlockSpec((1, SB, DH), lambda h, s: (h, s, 0)),
        out_shape=jax.ShapeDtypeStruct((H, nq * SB, DH), _bf16),
    )(q, k, v)


def _attn_call(q, k, v):
    nq = S // SB
    lo = _attn_half_call(q, k, v, skd=S // 2, sq_off=0, nq=nq // 2)
    hi = _attn_half_call(q, k, v, skd=S, sq_off=nq // 2, nq=nq // 2)
    return jnp.concatenate([lo, hi], axis=1)


def _projrouter_body(x_ref, o_ref, wo_ref, fnw_ref, rw_ref,
                     h_ref, zn_ref, w0_ref, w1_ref, s0_ref, s1_ref,
                     g0_ref, g1_ref, cnt_ref):
    @pl.when(pl.program_id(0) == 0)
    def _():
        cnt_ref[...] = jnp.zeros_like(cnt_ref)

    acc = x_ref[...]
    for h in range(H):
        acc = acc + _dot(o_ref[h], wo_ref[h])
    h_ref[...] = acc
    zn = acc * lax.rsqrt(jnp.mean(acc * acc, axis=1, keepdims=True) + 1e-6)
    zn = zn * fnw_ref[...]
    zn_ref[...] = zn
    lg = _dot(zn, rw_ref[...])
    io = lax.broadcasted_iota(_i32, (SB, E), 1)
    l1 = jnp.max(lg, axis=1, keepdims=True)
    e0 = jnp.min(jnp.where(lg == l1, io, E), axis=1, keepdims=True)
    lg2 = jnp.where(io == e0, -jnp.inf, lg)
    l2 = jnp.max(lg2, axis=1, keepdims=True)
    e1 = jnp.min(jnp.where(lg2 == l2, io, E), axis=1, keepdims=True)
    w0 = 1.0 / (1.0 + jnp.exp(l2 - l1))
    w1 = 1.0 - w0

    a0 = (io == e0).astype(_f32)
    a1 = (io == e1).astype(_f32)
    m2 = a0 + a1
    ri = lax.broadcasted_iota(_i32, (SB, SB), 0)
    ci = lax.broadcasted_iota(_i32, (SB, SB), 1)
    ltri = (ci <= ri).astype(_f32)
    incl = _dot(ltri, m2) + cnt_ref[0:1, :]
    cnt_ref[0:1, :] = incl[SB - 1:SB, :]

    pos0 = jnp.sum(incl * a0, axis=1, keepdims=True) - 1.0
    pos1 = jnp.sum(incl * a1, axis=1, keepdims=True) - 1.0
    keep0 = pos0 < C
    keep1 = pos1 < C
    p0i = pos0.astype(_i32)
    p1i = pos1.astype(_i32)
    pc0 = jnp.minimum(p0i, C - 1)
    pc1 = jnp.minimum(p1i, C - 1)

    def bcast(v):
        return jnp.broadcast_to(v, (SB, 128))

    w0_ref[...] = bcast(jnp.where(keep0, w0, 0.0))
    w1_ref[...] = bcast(jnp.where(keep1, w1, 0.0))
    s0_ref[...] = bcast(jnp.where(keep0, e0 * C + p0i, DUMMY))
    s1_ref[...] = bcast(jnp.where(keep1, e1 * C + p1i, DUMMY))
    g0_ref[...] = bcast(e0 * C + pc0)
    g1_ref[...] = bcast(e1 * C + pc1)


def _projrouter_call(x2, o, wo_r, fnw, rw):
    sd = jax.ShapeDtypeStruct((S, D), _f32)
    fs = jax.ShapeDtypeStruct((S, 128), _f32)
    is_ = jax.ShapeDtypeStruct((S, 128), _i32)
    return pl.pallas_call(
        _projrouter_body,
        grid=(S // SB,),
        in_specs=[
            pl.BlockSpec((SB, D), lambda s: (s, 0)),
            pl.BlockSpec((H, SB, DH), lambda s: (0, s, 0)),
            pl.BlockSpec((H, DH, D), lambda s: (0, 0, 0)),
            pl.BlockSpec((1, D), lambda s: (0, 0)),
            pl.BlockSpec((D, E), lambda s: (0, 0)),
        ],
        out_specs=([pl.BlockSpec((SB, D), lambda s: (s, 0))] * 2
                   + [pl.BlockSpec((SB, 128), lambda s: (s, 0))] * 6),
        out_shape=[sd, sd, fs, fs, is_, is_, is_, is_],
        scratch_shapes=[pltpu.VMEM((8, E), _f32)],
    )(x2, o, wo_r, fnw, rw)


@functools.cache
def _build_dispatch():
    mesh = plsc.VectorSubcoreMesh(core_axis_name="c", subcore_axis_name="s")

    @functools.partial(
        pl.kernel,
        mesh=mesh,
        out_type=jax.ShapeDtypeStruct((E * C + 128, D), _f32),
        scratch_types=[
            pltpu.VMEM((TPW,), _i32),
            pltpu.VMEM((TPW,), _i32),
            pltpu.VMEM((TPW, D), _f32),
            pltpu.SemaphoreType.DMA,
        ],
    )
    def disp(zn_hbm, s0_hbm, s1_hbm, out_hbm, i0_v, i1_v, rows_v, sem):
        wid = lax.axis_index("s") * NC + lax.axis_index("c")
        base = wid * TPW
        pltpu.sync_copy(s0_hbm.at[pl.ds(base, TPW)], i0_v)
        pltpu.sync_copy(s1_hbm.at[pl.ds(base, TPW)], i1_v)
        pltpu.sync_copy(zn_hbm.at[pl.ds(base, TPW)], rows_v)
        pltpu.async_copy(rows_v, out_hbm.at[i0_v], sem).wait()
        pltpu.async_copy(rows_v, out_hbm.at[i1_v], sem).wait()

    return disp


def _dispatch(zn, s0, s1):
    return _build_dispatch()(zn, s0, s1)


@functools.cache
def _build_combine():
    mesh = plsc.VectorSubcoreMesh(core_axis_name="c", subcore_axis_name="s")

    @functools.partial(
        pl.kernel,
        mesh=mesh,
        out_type=[jax.ShapeDtypeStruct((T, D), _f32),
                  jax.ShapeDtypeStruct((T, D), _f32)],
        scratch_types=[
            pltpu.VMEM((TPW,), _i32),
            pltpu.VMEM((TPW, D), _f32),
            pltpu.SemaphoreType.DMA,
        ],
    )
    def comb(oute_hbm, g0_hbm, g1_hbm, y0_hbm, y1_hbm, i_v, buf_v, sem):
        wid = lax.axis_index("s") * NC + lax.axis_index("c")
        base = wid * TPW
        pltpu.sync_copy(g0_hbm.at[pl.ds(base, TPW)], i_v)
        pltpu.async_copy(oute_hbm.at[i_v], buf_v, sem).wait()
        pltpu.sync_copy(buf_v, y0_hbm.at[pl.ds(base, TPW)])
        pltpu.sync_copy(g1_hbm.at[pl.ds(base, TPW)], i_v)
        pltpu.async_copy(oute_hbm.at[i_v], buf_v, sem).wait()
        pltpu.sync_copy(buf_v, y1_hbm.at[pl.ds(base, TPW)])

    return comb


def _combine(oute, g0, g1):
    return _build_combine()(oute, g0, g1)


def _ffn_body(in_ref, w1_ref, w3_ref, w2_ref, out_ref):
    xin = in_ref[...].astype(_bf16)
    g = _dot(xin, w1_ref[0].astype(_bf16))
    u = _dot(xin, w3_ref[0].astype(_bf16))
    a = g * (1.0 / (1.0 + jnp.exp(-g))) * u
    out_ref[...] = _dot(a.astype(_bf16), w2_ref[0].astype(_bf16))


def _ffn_call(ein, w1, w3, w2):
    return pl.pallas_call(
        _ffn_body,
        grid=(E,),
        in_specs=[
            pl.BlockSpec((C, D), lambda e: (e, 0)),
            pl.BlockSpec((1, D, F), lambda e: (e, 0, 0)),
            pl.BlockSpec((1, D, F), lambda e: (e, 0, 0)),
            pl.BlockSpec((1, F, D), lambda e: (e, 0, 0)),
        ],
        out_specs=pl.BlockSpec((C, D), lambda e: (e, 0)),
        out_shape=jax.ShapeDtypeStruct((E * C, D), _f32),
    )(ein, w1, w3, w2)


def _final_body(h_ref, y0_ref, y1_ref, w0_ref, w1_ref, o_ref):
    o_ref[...] = (h_ref[...]
                  + w0_ref[:, 0:1] * y0_ref[...]
                  + w1_ref[:, 0:1] * y1_ref[...])


def _final_call(h2, y0, y1, w0o, w1o):
    return pl.pallas_call(
        _final_body,
        grid=(S // SB,),
        in_specs=[
            pl.BlockSpec((SB, D), lambda s: (s, 0)),
            pl.BlockSpec((SB, D), lambda s: (s, 0)),
            pl.BlockSpec((SB, D), lambda s: (s, 0)),
            pl.BlockSpec((SB, 128), lambda s: (s, 0)),
            pl.BlockSpec((SB, 128), lambda s: (s, 0)),
        ],
        out_specs=pl.BlockSpec((SB, D), lambda s: (s, 0)),
        out_shape=jax.ShapeDtypeStruct((S, D), _f32),
    )(h2, y0, y1, w0o, w1o)


def kernel(x, attn_norm_w, Wq, Wk, Wv, Wo, ffn_norm_w, router_w, W1, W3, W2):
    x2 = x.reshape(S, D)
    wcat = jnp.concatenate([Wq, Wk, Wv], axis=1).astype(_bf16)
    wo_r = Wo.reshape(H, DH, D).astype(_bf16)
    anw = attn_norm_w.reshape(1, D)
    fnw = ffn_norm_w.reshape(1, D)

    inv_freq = 1.0 / (10000.0 ** (jnp.arange(0, HALF, dtype=_f32) / HALF))
    t = jnp.arange(S, dtype=_f32)
    freqs = jnp.outer(t, inv_freq)
    cosf = jnp.concatenate([jnp.cos(freqs)] * 2, axis=1)
    sinf = jnp.concatenate([jnp.sin(freqs)] * 2, axis=1)

    q, k, v = _qkv_call(x2, anw, wcat, cosf, sinf)
    o = _attn_call(q, k, v)
    (h2, zn, w0o, w1o, s0o, s1o, g0o, g1o) = _projrouter_call(
        x2, o, wo_r, fnw, router_w)

    s0 = s0o[:, 0]
    s1 = s1o[:, 0]
    g0 = g0o[:, 0]
    g1 = g1o[:, 0]

    ein = _dispatch(zn, s0, s1)
    oute = _ffn_call(ein, W1, W3, W2)
    y0, y1 = _combine(oute, g0, g1)

    out = _final_call(h2, y0, y1, w0o, w1o)
    return out.reshape(B, S, D)


# speedup vs baseline: 1.1498x; 1.0484x over previous
import functools

import jax
import jax.numpy as jnp
from jax import lax
from jax.experimental import pallas as pl
from jax.experimental.pallas import tpu as pltpu
from jax.experimental.pallas import tpu_sc as plsc

B, S, D, H = 1, 2048, 768, 12
DH = D // H
E, K, F, C = 64, 2, 512, 128
T = B * S
HALF = DH // 2

SB = 256
NC, NS = 2, 16
NW = NC * NS
TPW = T // NW
DUMMY = E * C

_f32 = jnp.float32
_i32 = jnp.int32
_bf16 = jnp.bfloat16


def _dot(a, b):
    return jnp.dot(a, b, preferred_element_type=_f32)


def _qkv_body(x_ref, anw_ref, wcat_ref, q_ref, k_ref, v_ref):
    xb = x_ref[...]
    hn = xb * lax.rsqrt(jnp.mean(xb * xb, axis=1, keepdims=True) + 1e-6)
    hn = hn * anw_ref[...]
    qkv = _dot(hn.astype(_bf16), wcat_ref[...])
    t = (lax.broadcasted_iota(_i32, (SB, HALF), 0)
         + pl.program_id(0) * SB).astype(_f32)
    j = lax.broadcasted_iota(_i32, (SB, HALF), 1).astype(_f32)
    freqs = t * jnp.exp(j * (-9.210340371976184 / HALF))
    cosb = jnp.concatenate([jnp.cos(freqs)] * 2, axis=1)
    sinb = jnp.concatenate([jnp.sin(freqs)] * 2, axis=1)

    def rope(t):
        t1 = t[:, :HALF]
        t2 = t[:, HALF:]
        rot = jnp.concatenate([-t2, t1], axis=1)
        return t * cosb + rot * sinb

    for h in range(H):
        qh = qkv[:, h * DH:(h + 1) * DH]
        kh = qkv[:, D + h * DH:D + (h + 1) * DH]
        vh = qkv[:, 2 * D + h * DH:2 * D + (h + 1) * DH]
        q_ref[h] = rope(qh).astype(_bf16)
        k_ref[h] = rope(kh).astype(_bf16)
        v_ref[h] = vh.astype(_bf16)


def _qkv_call(x2, anw, wcat):
    hsd = jax.ShapeDtypeStruct((H, S, DH), _bf16)
    return pl.pallas_call(
        _qkv_body,
        grid=(S // SB,),
        in_specs=[
            pl.BlockSpec((SB, D), lambda s: (s, 0)),
            pl.BlockSpec((1, D), lambda s: (0, 0)),
            pl.BlockSpec((D, 3 * D), lambda s: (0, 0)),
        ],
        out_specs=[pl.BlockSpec((H, SB, DH), lambda s: (0, s, 0))] * 3,
        out_shape=[hsd, hsd, hsd],
    )(x2, anw, wcat)


def _attn_body(q_ref, k_ref, v_ref, o_ref, *, skd, sq_off):
    sq = pl.program_id(1) + sq_off
    qb = q_ref[0]
    kb = k_ref[0]
    sc = lax.dot_general(qb, kb, (((1,), (1,)), ((), ())),
                         preferred_element_type=_f32) * (1.0 / 8.0)
    row = lax.broadcasted_iota(_i32, (SB, skd), 0) + sq * SB
    col = lax.broadcasted_iota(_i32, (SB, skd), 1)
    sc = jnp.where(col <= row, sc, -1e30)
    m = jnp.max(sc, axis=1, keepdims=True)
    p = jnp.exp(sc - m)
    l = jnp.sum(p, axis=1, keepdims=True)
    o_ref[0] = (_dot(p.astype(_bf16), v_ref[0]) / l).astype(_bf16)


def _attn_lo_call(q, k, v):
    skd = S // 2
    return pl.pallas_call(
        functools.partial(_attn_body, skd=skd, sq_off=0),
        grid=(H, S // SB // 2),
        in_specs=[
            pl.BlockSpec((1, SB, DH), lambda h, s: (h, s, 0)),
            pl.BlockSpec((1, skd, DH), lambda h, s: (h, 0, 0)),
            pl.BlockSpec((1, skd, DH), lambda h, s: (h, 0, 0)),
        ],
        out_specs=pl.BlockSpec((1, SB, DH), lambda h, s: (h, s, 0)),
        out_shape=jax.ShapeDtypeStruct((H, S, DH), _bf16),
    )(q, k, v)


def _attn_hi_call(q, k, v, o_lo):
    nqh = S // SB // 2

    def wrapped(o_in_ref, q_ref, k_ref, v_ref, o_ref):
        _attn_body(q_ref, k_ref, v_ref, o_ref, skd=S, sq_off=nqh)

    return pl.pallas_call(
        wrapped,
        grid=(H, nqh),
        in_specs=[
            pl.BlockSpec(memory_space=pl.ANY),
            pl.BlockSpec((1, SB, DH), lambda h, s: (h, s + nqh, 0)),
            pl.BlockSpec((1, S, DH), lambda h, s: (h, 0, 0)),
            pl.BlockSpec((1, S, DH), lambda h, s: (h, 0, 0)),
        ],
        out_specs=pl.BlockSpec((1, SB, DH), lambda h, s: (h, s + nqh, 0)),
        out_shape=jax.ShapeDtypeStruct((H, S, DH), _bf16),
        input_output_aliases={0: 0},
    )(o_lo, q, k, v)


def _attn_call(q, k, v):
    return _attn_hi_call(q, k, v, _attn_lo_call(q, k, v))


def _projrouter_body(x_ref, o_ref, wo_ref, fnw_ref, rw_ref,
                     h_ref, zn_ref, wf_ref, si_ref, cnt_ref):
    @pl.when(pl.program_id(0) == 0)
    def _():
        cnt_ref[...] = jnp.zeros_like(cnt_ref)

    acc = x_ref[...]
    for h in range(H):
        acc = acc + _dot(o_ref[h], wo_ref[h])
    h_ref[...] = acc
    zn = acc * lax.rsqrt(jnp.mean(acc * acc, axis=1, keepdims=True) + 1e-6)
    zn = zn * fnw_ref[...]
    zn_ref[...] = zn
    lg = _dot(zn, rw_ref[...])
    io = lax.broadcasted_iota(_i32, (SB, E), 1)
    l1 = jnp.max(lg, axis=1, keepdims=True)
    e0 = jnp.min(jnp.where(lg == l1, io, E), axis=1, keepdims=True)
    lg2 = jnp.where(io == e0, -jnp.inf, lg)
    l2 = jnp.max(lg2, axis=1, keepdims=True)
    e1 = jnp.min(jnp.where(lg2 == l2, io, E), axis=1, keepdims=True)
    w0 = 1.0 / (1.0 + jnp.exp(l2 - l1))
    w1 = 1.0 - w0

    a0 = (io == e0).astype(_f32)
    a1 = (io == e1).astype(_f32)
    m2 = a0 + a1
    ri = lax.broadcasted_iota(_i32, (SB, SB), 0)
    ci = lax.broadcasted_iota(_i32, (SB, SB), 1)
    ltri = (ci <= ri).astype(_f32)
    incl = _dot(ltri, m2) + cnt_ref[0:1, :]
    cnt_ref[0:1, :] = incl[SB - 1:SB, :]

    pos0 = jnp.sum(incl * a0, axis=1, keepdims=True) - 1.0
    pos1 = jnp.sum(incl * a1, axis=1, keepdims=True) - 1.0
    keep0 = pos0 < C
    keep1 = pos1 < C
    p0i = pos0.astype(_i32)
    p1i = pos1.astype(_i32)
    pc0 = jnp.minimum(p0i, C - 1)
    pc1 = jnp.minimum(p1i, C - 1)

    w0k = jnp.where(keep0, w0, 0.0)
    w1k = jnp.where(keep1, w1, 0.0)
    wf_ref[...] = jnp.concatenate([w0k, w1k, w0k, w1k], axis=1)
    si_ref[...] = jnp.concatenate(
        [jnp.where(keep0, e0 * C + p0i, DUMMY),
         jnp.where(keep1, e1 * C + p1i, DUMMY),
         e0 * C + pc0,
         e1 * C + pc1], axis=1)


def _projrouter_call(x2, o, wo_r, fnw, rw):
    sd = jax.ShapeDtypeStruct((S, D), _f32)
    return pl.pallas_call(
        _projrouter_body,
        grid=(S // SB,),
        in_specs=[
            pl.BlockSpec((SB, D), lambda s: (s, 0)),
            pl.BlockSpec((H, SB, DH), lambda s: (0, s, 0)),
            pl.BlockSpec((H, DH, D), lambda s: (0, 0, 0)),
            pl.BlockSpec((1, D), lambda s: (0, 0)),
            pl.BlockSpec((D, E), lambda s: (0, 0)),
        ],
        out_specs=([pl.BlockSpec((SB, D), lambda s: (s, 0))] * 2
                   + [pl.BlockSpec((SB, 4), lambda s: (s, 0))] * 2),
        out_shape=[sd, sd, jax.ShapeDtypeStruct((S, 4), _f32),
                   jax.ShapeDtypeStruct((S, 4), _i32)],
        scratch_shapes=[pltpu.VMEM((8, E), _f32)],
    )(x2, o, wo_r, fnw, rw)


@functools.cache
def _build_dispatch():
    mesh = plsc.VectorSubcoreMesh(core_axis_name="c", subcore_axis_name="s")

    @functools.partial(
        pl.kernel,
        mesh=mesh,
        out_type=jax.ShapeDtypeStruct((E * C + 256, D), _f32),
        scratch_types=[
            pltpu.VMEM((TPW,), _i32),
            pltpu.VMEM((TPW,), _i32),
            pltpu.VMEM((TPW, D), _f32),
            pltpu.SemaphoreType.DMA,
            pltpu.SemaphoreType.DMA,
            pltpu.SemaphoreType.DMA,
            pltpu.SemaphoreType.DMA,
            pltpu.SemaphoreType.DMA,
        ],
    )
    def disp(zn_hbm, s0_hbm, s1_hbm, out_hbm, i0_v, i1_v, rows_v,
             m1, m2, m3, m4, m5):
        wid = lax.axis_index("s") * NC + lax.axis_index("c")
        base = wid * TPW
        a = pltpu.async_copy(s0_hbm.at[pl.ds(base, TPW)], i0_v, m1)
        b = pltpu.async_copy(s1_hbm.at[pl.ds(base, TPW)], i1_v, m2)
        c = pltpu.async_copy(zn_hbm.at[pl.ds(base, TPW)], rows_v, m3)
        a.wait()
        c.wait()
        sa = pltpu.async_copy(rows_v, out_hbm.at[i0_v], m4)
        b.wait()
        sb = pltpu.async_copy(rows_v, out_hbm.at[i1_v], m5)
        sa.wait()
        sb.wait()

    return disp


def _dispatch(zn, s0, s1):
    return _build_dispatch()(zn, s0, s1)


@functools.cache
def _build_combine():
    mesh = plsc.VectorSubcoreMesh(core_axis_name="c", subcore_axis_name="s")

    @functools.partial(
        pl.kernel,
        mesh=mesh,
        out_type=[jax.ShapeDtypeStruct((T, D), _f32),
                  jax.ShapeDtypeStruct((T, D), _f32)],
        scratch_types=[
            pltpu.VMEM((TPW,), _i32),
            pltpu.VMEM((TPW,), _i32),
            pltpu.VMEM((TPW, D), _f32),
            pltpu.VMEM((TPW, D), _f32),
            pltpu.SemaphoreType.DMA,
            pltpu.SemaphoreType.DMA,
            pltpu.SemaphoreType.DMA,
            pltpu.SemaphoreType.DMA,
        ],
    )
    def comb(oute_hbm, g0_hbm, g1_hbm, y0_hbm, y1_hbm,
             i0_v, i1_v, buf0_v, buf1_v, m1, m2, m3, m4):
        wid = lax.axis_index("s") * NC + lax.axis_index("c")
        base = wid * TPW
        a = pltpu.async_copy(g0_hbm.at[pl.ds(base, TPW)], i0_v, m1)
        b = pltpu.async_copy(g1_hbm.at[pl.ds(base, TPW)], i1_v, m2)
        a.wait()
        ga = pltpu.async_copy(oute_hbm.at[i0_v], buf0_v, m3)
        b.wait()
        gb = pltpu.async_copy(oute_hbm.at[i1_v], buf1_v, m4)
        ga.wait()
        pltpu.sync_copy(buf0_v, y0_hbm.at[pl.ds(base, TPW)])
        gb.wait()
        pltpu.sync_copy(buf1_v, y1_hbm.at[pl.ds(base, TPW)])

    return comb


def _combine(oute, g0, g1):
    return _build_combine()(oute, g0, g1)


EPB = 2


def _ffn_body(in_ref, w1_ref, w3_ref, w2_ref, out_ref):
    for i in range(EPB):
        xin = in_ref[i * C:(i + 1) * C].astype(_bf16)
        g = _dot(xin, w1_ref[i].astype(_bf16))
        u = _dot(xin, w3_ref[i].astype(_bf16))
        a = g * (1.0 / (1.0 + jnp.exp(-g))) * u
        out_ref[i * C:(i + 1) * C] = _dot(a.astype(_bf16),
                                          w2_ref[i].astype(_bf16))


def _ffn_call(ein, w1, w3, w2):
    return pl.pallas_call(
        _ffn_body,
        grid=(E // EPB,),
        in_specs=[
            pl.BlockSpec((EPB * C, D), lambda e: (e, 0)),
            pl.BlockSpec((EPB, D, F), lambda e: (e, 0, 0)),
            pl.BlockSpec((EPB, D, F), lambda e: (e, 0, 0)),
            pl.BlockSpec((EPB, F, D), lambda e: (e, 0, 0)),
        ],
        out_specs=pl.BlockSpec((EPB * C, D), lambda e: (e, 0)),
        out_shape=jax.ShapeDtypeStruct((E * C, D), _f32),
    )(ein, w1, w3, w2)


def _final_body(h_ref, y0_ref, y1_ref, wf_ref, o_ref):
    o_ref[...] = (h_ref[...]
                  + wf_ref[:, 0:1] * y0_ref[...]
                  + wf_ref[:, 1:2] * y1_ref[...])


def _final_call(h2, y0, y1, wf):
    return pl.pallas_call(
        _final_body,
        grid=(S // SB,),
        in_specs=[
            pl.BlockSpec((SB, D), lambda s: (s, 0)),
            pl.BlockSpec((SB, D), lambda s: (s, 0)),
            pl.BlockSpec((SB, D), lambda s: (s, 0)),
            pl.BlockSpec((SB, 4), lambda s: (s, 0)),
        ],
        out_specs=pl.BlockSpec((SB, D), lambda s: (s, 0)),
        out_shape=jax.ShapeDtypeStruct((S, D), _f32),
    )(h2, y0, y1, wf)


def kernel(x, attn_norm_w, Wq, Wk, Wv, Wo, ffn_norm_w, router_w, W1, W3, W2):
    x2 = x.reshape(S, D)
    wcat = jnp.concatenate([Wq, Wk, Wv], axis=1).astype(_bf16)
    wo_r = Wo.reshape(H, DH, D).astype(_bf16)
    anw = attn_norm_w.reshape(1, D)
    fnw = ffn_norm_w.reshape(1, D)

    q, k, v = _qkv_call(x2, anw, wcat)
    o = _attn_call(q, k, v)
    h2, zn, wf, si = _projrouter_call(x2, o, wo_r, fnw, router_w)

    s0 = si[:, 0]
    s1 = si[:, 1]
    g0 = si[:, 2]
    g1 = si[:, 3]

    ein = _dispatch(zn, s0, s1)
    oute = _ffn_call(ein, W1, W3, W2)
    y0, y1 = _combine(oute, g0, g1)

    out = _final_call(h2, y0, y1, wf)
    return out.reshape(B, S, D)
